# baseline (speedup 1.0000x reference)
import functools

import jax
import jax.numpy as jnp
from jax import lax
from jax.experimental import pallas as pl
from jax.experimental.pallas import tpu as pltpu
from jax.experimental.pallas import tpu_sc as plsc

BATCH = 4096
HIST = 200
EMBED = 64
VOCAB = 1000000
SPLIT = 104

NBUF = 4
RB = 8192
HB = RB // 2
HBLOG = HB.bit_length() - 1
N_BLOCKS = (VOCAB + RB - 1) // RB
VOCAB_PAD = N_BLOCKS * RB


def _repack_kernel(tt_ref, eye_ref, out_ref):
  def tr(sl):
    return jax.lax.dot_general(
        tt_ref[:, sl], eye_ref[...], (((0,), (0,)), ((), ())),
        precision=jax.lax.Precision.HIGHEST,
        preferred_element_type=jnp.float32)

  out_ref[...] = jnp.concatenate([tr(pl.ds(0, HB)), tr(pl.ds(HB, HB))],
                                 axis=1)


def _repack(table_t):
  eye = jnp.eye(EMBED, EMBED, dtype=jnp.float32)
  return pl.pallas_call(
      _repack_kernel,
      grid=(N_BLOCKS,),
      in_specs=[
          pl.BlockSpec((EMBED, RB), lambda c: (0, c)),
          pl.BlockSpec((EMBED, EMBED), lambda c: (0, 0)),
      ],
      out_specs=pl.BlockSpec((HB, 2 * EMBED), lambda c: (c, 0)),
      out_shape=jax.ShapeDtypeStruct((VOCAB_PAD // 2, 2 * EMBED), jnp.float32),
  )(table_t, eye)


def _make_pooling_kernel():
  info = plsc.get_sparse_core_info()
  nw = info.num_cores * info.num_subcores
  b_per_w = BATCH // nw

  mesh = plsc.VectorSubcoreMesh(core_axis_name="c", subcore_axis_name="s")

  @functools.partial(
      pl.kernel,
      mesh=mesh,
      compiler_params=pltpu.CompilerParams(use_tc_tiling_on_sc=False),
      out_type=jax.ShapeDtypeStruct((BATCH * EMBED,), jnp.float32),
      scratch_types=[
          pltpu.VMEM((b_per_w * HIST,), jnp.int32),
          pltpu.VMEM((NBUF, HIST, EMBED), jnp.float32),
          pltpu.VMEM((b_per_w * EMBED,), jnp.float32),
          [pltpu.SemaphoreType.DMA] * NBUF,
      ],
  )
  def pool(x_hbm, table_hbm, out_hbm, idx_v, rows_v, pooled_v, sems):
    wid = lax.axis_index("s") * info.num_cores + lax.axis_index("c")
    base = wid * b_per_w

    pltpu.sync_copy(x_hbm.at[pl.ds(base * HIST, b_per_w * HIST)], idx_v)

    def remap_body(k, _):
      o = k * 128
      for u in range(8):
        v = idx_v[pl.ds(o + u * 16, 16)]
        idx_v[pl.ds(o + u * 16, 16)] = (
            (v & (-RB)) | ((v & (HB - 1)) << 1) | ((v & HB) >> HBLOG))
      return ()

    lax.fori_loop(0, b_per_w * HIST // 128, remap_body, ())

    def fire(b, p):
      pltpu.async_copy(
          table_hbm.at[idx_v.at[pl.ds(b * HIST, SPLIT)]],
          rows_v.at[p, pl.ds(0, SPLIT)], sems[p])
      pltpu.async_copy(
          table_hbm.at[idx_v.at[pl.ds(b * HIST + SPLIT, HIST - SPLIT)]],
          rows_v.at[p, pl.ds(SPLIT, HIST - SPLIT)], sems[p])

    def consume(b, p):
      pltpu.make_async_copy(
          table_hbm.at[pl.ds(0, HIST)], rows_v.at[p], sems[p]).wait()

      def sum_body(i, acc):
        a0, a1, a2, a3 = acc
        l0 = i * 8
        for u in range(8):
          a0 = a0 + rows_v[p, l0 + u, pl.ds(0, 16)]
          a1 = a1 + rows_v[p, l0 + u, pl.ds(16, 16)]
          a2 = a2 + rows_v[p, l0 + u, pl.ds(32, 16)]
          a3 = a3 + rows_v[p, l0 + u, pl.ds(48, 16)]
        return (a0, a1, a2, a3)

      zero = jnp.zeros((16,), jnp.float32)
      a0, a1, a2, a3 = lax.fori_loop(
          0, HIST // 8, sum_body, (zero, zero, zero, zero))
      pooled_v[pl.ds(b * EMBED, 16)] = a0
      pooled_v[pl.ds(b * EMBED + 16, 16)] = a1
      pooled_v[pl.ds(b * EMBED + 32, 16)] = a2
      pooled_v[pl.ds(b * EMBED + 48, 16)] = a3

    for p in range(NBUF):
      fire(p, p)

    def group_body(g, _):
      for p in range(NBUF):
        b = g * NBUF + p
        consume(b, p)
        fire(b + NBUF, p)
      return ()

    n_full = (b_per_w - NBUF) // NBUF
    lax.fori_loop(0, n_full, group_body, ())

    for b in range(n_full * NBUF, b_per_w):
      consume(b, b % NBUF)
      if b + NBUF < b_per_w:
        fire(b + NBUF, b % NBUF)

    pltpu.sync_copy(pooled_v, out_hbm.at[pl.ds(base * EMBED, b_per_w * EMBED)])

  return pool


_pooling_kernel = _make_pooling_kernel()


def _mlp_kernel(pooled_ref, w1_ref, b1_ref, w2_ref, b2_ref, out_ref):
  h = jnp.dot(pooled_ref[...], w1_ref[...],
              preferred_element_type=jnp.float32)
  h = jnp.maximum(h + b1_ref[...], 0.0)
  out_ref[...] = jnp.dot(h, w2_ref[...],
                         preferred_element_type=jnp.float32) + b2_ref[...]


@jax.jit
def kernel(x, table, W1, b1, W2, b2):
  table_lin = _repack(table.T).reshape(VOCAB_PAD, EMBED)
  pooled = _pooling_kernel(x.reshape(-1), table_lin).reshape(BATCH, EMBED)
  out = pl.pallas_call(
      _mlp_kernel,
      out_shape=jax.ShapeDtypeStruct((BATCH, 4), jnp.float32),
  )(pooled, W1, b1.reshape(1, 100), W2, b2.reshape(1, 4))
  return out


# speedup vs baseline: 1.1027x; 1.1027x over previous
import functools

import jax
import jax.numpy as jnp
from jax import lax
from jax.experimental import pallas as pl
from jax.experimental.pallas import tpu as pltpu
from jax.experimental.pallas import tpu_sc as plsc

BATCH = 4096
HIST = 200
EMBED = 64
VOCAB = 1000000
SPLIT = 104

NBUF = 4
RB = 8192
HB = RB // 2
HBLOG = HB.bit_length() - 1
N_BLOCKS = (VOCAB + RB - 1) // RB
VOCAB_PAD = N_BLOCKS * RB


def _repack_kernel(tt_ref, eye_ref, out_ref):
  r = jax.lax.dot_general(
      tt_ref[...], eye_ref[...], (((0,), (0,)), ((), ())),
      precision=jax.lax.Precision.HIGHEST,
      preferred_element_type=jnp.float32)
  top = jax.lax.slice(r, (0, 0), (HB, EMBED))
  bot = jax.lax.slice(r, (HB, 0), (RB, EMBED))
  out_ref[...] = jnp.concatenate([top, bot], axis=1)


def _repack(table_t):
  eye = jnp.eye(EMBED, EMBED, dtype=jnp.float32)
  return pl.pallas_call(
      _repack_kernel,
      grid=(N_BLOCKS,),
      in_specs=[
          pl.BlockSpec((EMBED, RB), lambda c: (0, c)),
          pl.BlockSpec((EMBED, EMBED), lambda c: (0, 0)),
      ],
      out_specs=pl.BlockSpec((HB, 2 * EMBED), lambda c: (c, 0)),
      out_shape=jax.ShapeDtypeStruct((VOCAB_PAD // 2, 2 * EMBED), jnp.float32),
  )(table_t, eye)


def _make_pooling_kernel():
  info = plsc.get_sparse_core_info()
  nw = info.num_cores * info.num_subcores
  b_per_w = BATCH // nw

  mesh = plsc.VectorSubcoreMesh(core_axis_name="c", subcore_axis_name="s")

  @functools.partial(
      pl.kernel,
      mesh=mesh,
      compiler_params=pltpu.CompilerParams(use_tc_tiling_on_sc=False),
      out_type=jax.ShapeDtypeStruct((BATCH * EMBED,), jnp.float32),
      scratch_types=[
          pltpu.VMEM((b_per_w * HIST,), jnp.int32),
          pltpu.VMEM((NBUF, HIST, EMBED), jnp.float32),
          pltpu.VMEM((b_per_w * EMBED,), jnp.float32),
          [pltpu.SemaphoreType.DMA] * NBUF,
      ],
  )
  def pool(x_hbm, table_hbm, out_hbm, idx_v, rows_v, pooled_v, sems):
    wid = lax.axis_index("s") * info.num_cores + lax.axis_index("c")
    base = wid * b_per_w

    pltpu.sync_copy(x_hbm.at[pl.ds(base * HIST, b_per_w * HIST)], idx_v)

    def remap_body(k, _):
      o = k * 128
      for u in range(8):
        v = idx_v[pl.ds(o + u * 16, 16)]
        idx_v[pl.ds(o + u * 16, 16)] = (
            (v & (-RB)) | ((v & (HB - 1)) << 1) | ((v & HB) >> HBLOG))
      return ()

    lax.fori_loop(0, b_per_w * HIST // 128, remap_body, ())

    def fire(b, p):
      pltpu.async_copy(
          table_hbm.at[idx_v.at[pl.ds(b * HIST, SPLIT)]],
          rows_v.at[p, pl.ds(0, SPLIT)], sems[p])
      pltpu.async_copy(
          table_hbm.at[idx_v.at[pl.ds(b * HIST + SPLIT, HIST - SPLIT)]],
          rows_v.at[p, pl.ds(SPLIT, HIST - SPLIT)], sems[p])

    def consume(b, p):
      pltpu.make_async_copy(
          table_hbm.at[pl.ds(0, HIST)], rows_v.at[p], sems[p]).wait()

      def sum_body(i, acc):
        a0, a1, a2, a3 = acc
        l0 = i * 8
        for u in range(8):
          a0 = a0 + rows_v[p, l0 + u, pl.ds(0, 16)]
          a1 = a1 + rows_v[p, l0 + u, pl.ds(16, 16)]
          a2 = a2 + rows_v[p, l0 + u, pl.ds(32, 16)]
          a3 = a3 + rows_v[p, l0 + u, pl.ds(48, 16)]
        return (a0, a1, a2, a3)

      zero = jnp.zeros((16,), jnp.float32)
      a0, a1, a2, a3 = lax.fori_loop(
          0, HIST // 8, sum_body, (zero, zero, zero, zero))
      pooled_v[pl.ds(b * EMBED, 16)] = a0
      pooled_v[pl.ds(b * EMBED + 16, 16)] = a1
      pooled_v[pl.ds(b * EMBED + 32, 16)] = a2
      pooled_v[pl.ds(b * EMBED + 48, 16)] = a3

    for p in range(NBUF):
      fire(p, p)

    def group_body(g, _):
      for p in range(NBUF):
        b = g * NBUF + p
        consume(b, p)
        fire(b + NBUF, p)
      return ()

    n_full = (b_per_w - NBUF) // NBUF
    lax.fori_loop(0, n_full, group_body, ())

    for b in range(n_full * NBUF, b_per_w):
      consume(b, b % NBUF)
      if b + NBUF < b_per_w:
        fire(b + NBUF, b % NBUF)

    pltpu.sync_copy(pooled_v, out_hbm.at[pl.ds(base * EMBED, b_per_w * EMBED)])

  return pool


_pooling_kernel = _make_pooling_kernel()


def _mlp_kernel(pooled_ref, w1_ref, b1_ref, w2_ref, b2_ref, out_ref):
  h = jnp.dot(pooled_ref[...], w1_ref[...],
              preferred_element_type=jnp.float32)
  h = jnp.maximum(h + b1_ref[...], 0.0)
  out_ref[...] = jnp.dot(h, w2_ref[...],
                         preferred_element_type=jnp.float32) + b2_ref[...]


@jax.jit
def kernel(x, table, W1, b1, W2, b2):
  table_lin = _repack(table.T).reshape(VOCAB_PAD, EMBED)
  pooled = _pooling_kernel(x.reshape(-1), table_lin).reshape(BATCH, EMBED)
  out = pl.pallas_call(
      _mlp_kernel,
      out_shape=jax.ShapeDtypeStruct((BATCH, 4), jnp.float32),
  )(pooled, W1, b1.reshape(1, 100), W2, b2.reshape(1, 4))
  return out
